# baseline (speedup 1.0000x reference)
import functools

import jax
import jax.numpy as jnp
from jax import lax
from jax.experimental import pallas as pl
from jax.experimental.pallas import tpu as pltpu
from jax.experimental.pallas import tpu_sc as plsc

E = 8
TOPK = 2
D = 1024
DFF = 4096
M = 512
NT = 15
NROWS = NT * M
F_BLK = 1024
K1 = DFF // F_BLK

_INTERPRET = False

_NC = 2
_NS = 16
_NW = _NC * _NS
_T = 2048
_RPW = (NT * M) // _NW
_GCH = _RPW // 2
_TPW = _T // _NW
_CCH = 32


@functools.cache
def _sc_kernels():
    mesh1 = plsc.VectorSubcoreMesh(core_axis_name="c", subcore_axis_name="s")
    mesh2 = plsc.VectorSubcoreMesh(core_axis_name="c", subcore_axis_name="s")

    @functools.partial(
        pl.kernel,
        mesh=mesh1,
        out_type=jax.ShapeDtypeStruct((NT * M, D // 2), jnp.int32),
        scratch_types=[
            pltpu.VMEM((2, _GCH), jnp.int32),
            pltpu.VMEM((_RPW, D // 2), jnp.int32),
            pltpu.SemaphoreType.DMA,
        ],
    )
    def sc_gather(xbf_hbm, gidx_hbm, out_hbm, idx_v, rows_v, sem):
        wid = lax.axis_index("s") * _NC + lax.axis_index("c")
        base = wid * _RPW
        pltpu.sync_copy(gidx_hbm.at[wid], idx_v)
        cps = [
            pltpu.async_copy(xbf_hbm.at[idx_v.at[j]],
                             rows_v.at[pl.ds(j * _GCH, _GCH)], sem)
            for j in range(2)
        ]
        for cp in cps:
            cp.wait()
        pltpu.sync_copy(rows_v, out_hbm.at[pl.ds(base, _RPW)])

    @functools.partial(
        pl.kernel,
        mesh=mesh2,
        out_type=jax.ShapeDtypeStruct((_T, D), jnp.float32),
        scratch_types=[
            pltpu.VMEM((2, _CCH), jnp.int32),
            pltpu.VMEM((_CCH, D), jnp.float32),
            pltpu.VMEM((_CCH, D), jnp.float32),
            pltpu.VMEM((_CCH, D), jnp.float32),
            pltpu.SemaphoreType.DMA,
        ],
    )
    def sc_combine(y_hbm, pos_hbm, h_hbm, out_hbm, idx_v, y1_v, y2_v, h_v,
                   sem):
        wid = lax.axis_index("s") * _NC + lax.axis_index("c")
        for c in range(_TPW // _CCH):
            base = wid * _TPW + c * _CCH
            pltpu.sync_copy(pos_hbm.at[0, pl.ds(base, _CCH)], idx_v.at[0])
            pltpu.sync_copy(pos_hbm.at[1, pl.ds(base, _CCH)], idx_v.at[1])
            pltpu.sync_copy(h_hbm.at[pl.ds(base, _CCH)], h_v)
            cp1 = pltpu.async_copy(y_hbm.at[idx_v.at[0]], y1_v, sem)
            cp2 = pltpu.async_copy(y_hbm.at[idx_v.at[1]], y2_v, sem)
            cp1.wait()
            cp2.wait()

            def _row(r, _):
                for j in range(D // 16):
                    sl = pl.ds(j * 16, 16)
                    h_v[r, sl] = h_v[r, sl] + y1_v[r, sl] + y2_v[r, sl]
                return 0

            lax.fori_loop(0, _CCH, _row, 0)
            pltpu.sync_copy(h_v, out_hbm.at[pl.ds(base, _CCH)])

    return sc_gather, sc_combine


def _gmm1_body(te_ref, x_ref, wg_ref, wu_ref, ws_ref, act_ref):
    t = pl.program_id(1)
    x = x_ref[pl.ds(t * M, M), :]
    wg = wg_ref[0].astype(jnp.bfloat16)
    wu = wu_ref[0].astype(jnp.bfloat16)
    g = lax.dot_general(x, wg, (((1,), (1,)), ((), ())),
                        preferred_element_type=jnp.float32)
    u = lax.dot_general(x, wu, (((1,), (1,)), ((), ())),
                        preferred_element_type=jnp.float32)
    act = (g * jax.nn.sigmoid(g)) * u
    w = ws_ref[0, 0, :]
    act_ref[...] = (act * w[:, None]).astype(jnp.bfloat16)


def _gmm2_body(te_ref, act_ref, wd_ref, out_ref):
    wd = wd_ref[0].astype(jnp.bfloat16)
    out_ref[...] = lax.dot_general(act_ref[...], wd,
                                   (((1,), (1,)), ((), ())),
                                   preferred_element_type=jnp.float32)


def kernel(hidden_states, residual, ln_weight, ln_bias, router_weight,
           gate_up_proj, down_proj):
    B, S, _ = hidden_states.shape
    T = B * S

    h3 = residual + hidden_states
    mu = jnp.mean(h3, axis=-1, keepdims=True)
    var = jnp.mean((h3 - mu) ** 2, axis=-1, keepdims=True)
    normed = (h3 - mu) * lax.rsqrt(var + 1e-5) * ln_weight + ln_bias
    hidden_2d = normed.reshape(T, D)
    logits = hidden_2d @ router_weight.T
    probs = jax.nn.softmax(logits, axis=-1)
    routing_weights, selected_experts = lax.top_k(probs, TOPK)
    routing_weights = routing_weights / jnp.sum(routing_weights, axis=-1,
                                                keepdims=True)

    eflat = selected_experts.reshape(-1).astype(jnp.int32)
    rwflat = routing_weights.reshape(-1)
    oh = jax.nn.one_hot(eflat, E, dtype=jnp.int32)
    csum = jnp.cumsum(oh, axis=0)
    counts = csum[-1]
    rank = jnp.take_along_axis(csum, eflat[:, None], axis=1)[:, 0] - 1
    ntiles = (counts + M - 1) // M
    tile_end = jnp.cumsum(ntiles)
    tile_start = tile_end - ntiles
    row_start = tile_start * M
    pos = jnp.take(row_start, eflat) + rank
    tile_ids = jnp.arange(NT, dtype=jnp.int32)
    te = jnp.searchsorted(tile_end, tile_ids, side='right').astype(jnp.int32)
    te = jnp.minimum(te, E - 1)
    gidx = jnp.zeros((NROWS,), jnp.int32).at[pos].set(
        jnp.arange(2 * T, dtype=jnp.int32) // TOPK)
    ws = jnp.zeros((NROWS,), jnp.float32).at[pos].set(rwflat)

    xbf = hidden_2d.astype(jnp.bfloat16)
    xi = lax.bitcast_convert_type(xbf.reshape(T, D // 2, 2),
                                  jnp.int32)
    sc_gather, sc_combine = _sc_kernels()
    xs_i = sc_gather(xi, gidx.reshape(_NW, 2, _GCH))
    x_sorted = lax.bitcast_convert_type(
        xs_i, jnp.bfloat16).reshape(NROWS, D)

    ws3 = ws.reshape(NT, 1, M)
    act = pl.pallas_call(
        _gmm1_body,
        grid_spec=pltpu.PrefetchScalarGridSpec(
            num_scalar_prefetch=1,
            grid=(K1, NT),
            in_specs=[
                pl.BlockSpec((NROWS, D), lambda k, t, te: (0, 0)),
                pl.BlockSpec((1, F_BLK, D), lambda k, t, te: (te[t], k, 0)),
                pl.BlockSpec((1, F_BLK, D),
                             lambda k, t, te: (te[t], K1 + k, 0)),
                pl.BlockSpec((1, 1, M), lambda k, t, te: (t, 0, 0)),
            ],
            out_specs=pl.BlockSpec((M, F_BLK), lambda k, t, te: (t, k)),
        ),
        out_shape=jax.ShapeDtypeStruct((NROWS, DFF), jnp.bfloat16),
        interpret=_INTERPRET,
    )(te, x_sorted, gate_up_proj, gate_up_proj, ws3)

    y_sorted = pl.pallas_call(
        _gmm2_body,
        grid_spec=pltpu.PrefetchScalarGridSpec(
            num_scalar_prefetch=1,
            grid=(NT,),
            in_specs=[
                pl.BlockSpec((M, DFF), lambda t, te: (t, 0)),
                pl.BlockSpec((1, D, DFF), lambda t, te: (te[t], 0, 0)),
            ],
            out_specs=pl.BlockSpec((M, D), lambda t, te: (t, 0)),
        ),
        out_shape=jax.ShapeDtypeStruct((NROWS, D), jnp.float32),
        interpret=_INTERPRET,
    )(te, act, down_proj)

    pos2 = pos.reshape(T, TOPK).T.astype(jnp.int32)
    out = sc_combine(y_sorted, pos2, h3.reshape(T, D))
    return out.reshape(B, S, D)


# speedup vs baseline: 1.6291x; 1.6291x over previous
import functools

import jax
import jax.numpy as jnp
from jax import lax
from jax.experimental import pallas as pl
from jax.experimental.pallas import tpu as pltpu
from jax.experimental.pallas import tpu_sc as plsc

E = 8
TOPK = 2
D = 1024
DFF = 4096
M = 512
NT = 15
NROWS = NT * M
F_BLK = 1024
K1 = DFF // F_BLK

_NC = 2
_NS = 16
_NW = _NC * _NS
_T = 2048
_TPW = _T // _NW
_CCH = 32


@functools.cache
def _sc_kernels():
    mesh = plsc.VectorSubcoreMesh(core_axis_name="c", subcore_axis_name="s")

    @functools.partial(
        pl.kernel,
        mesh=mesh,
        out_type=jax.ShapeDtypeStruct((_T, D), jnp.float32),
        scratch_types=[
            pltpu.VMEM((2, _CCH), jnp.int32),
            pltpu.VMEM((_CCH, D), jnp.float32),
            pltpu.VMEM((_CCH, D), jnp.float32),
            pltpu.VMEM((_CCH, D), jnp.float32),
            pltpu.SemaphoreType.DMA,
        ],
    )
    def sc_combine(y_hbm, pos_hbm, h_hbm, out_hbm, idx_v, y1_v, y2_v, h_v,
                   sem):
        wid = lax.axis_index("s") * _NC + lax.axis_index("c")
        for c in range(_TPW // _CCH):
            base = wid * _TPW + c * _CCH
            pltpu.sync_copy(pos_hbm.at[0, pl.ds(base, _CCH)], idx_v.at[0])
            pltpu.sync_copy(pos_hbm.at[1, pl.ds(base, _CCH)], idx_v.at[1])
            pltpu.sync_copy(h_hbm.at[pl.ds(base, _CCH)], h_v)
            cp1 = pltpu.async_copy(y_hbm.at[idx_v.at[0]], y1_v, sem)
            cp2 = pltpu.async_copy(y_hbm.at[idx_v.at[1]], y2_v, sem)
            cp1.wait()
            cp2.wait()

            def _row(r, _):
                for j in range(D // 16):
                    sl = pl.ds(j * 16, 16)
                    h_v[r, sl] = h_v[r, sl] + y1_v[r, sl] + y2_v[r, sl]
                return 0

            lax.fori_loop(0, _CCH, _row, 0)
            pltpu.sync_copy(h_v, out_hbm.at[pl.ds(base, _CCH)])

    return sc_combine


def _gmm1_body(te_ref, x_ref, wg_ref, wu_ref, ws_ref, act_ref):
    t = pl.program_id(1)
    x = x_ref[pl.ds(t * M, M), :]
    wg = wg_ref[0].astype(jnp.bfloat16)
    wu = wu_ref[0].astype(jnp.bfloat16)
    g = lax.dot_general(x, wg, (((1,), (1,)), ((), ())),
                        preferred_element_type=jnp.float32)
    u = lax.dot_general(x, wu, (((1,), (1,)), ((), ())),
                        preferred_element_type=jnp.float32)
    act = (g * jax.nn.sigmoid(g)) * u
    w = ws_ref[0, 0, :]
    act_ref[...] = (act * w[:, None]).astype(jnp.bfloat16)


def _gmm2_body(te_ref, act_ref, wd_ref, out_ref):
    wd = wd_ref[0].astype(jnp.bfloat16)
    out_ref[...] = lax.dot_general(act_ref[...], wd,
                                   (((1,), (1,)), ((), ())),
                                   preferred_element_type=jnp.float32)


def kernel(hidden_states, residual, ln_weight, ln_bias, router_weight,
           gate_up_proj, down_proj):
    B, S, _ = hidden_states.shape
    T = B * S

    h3 = residual + hidden_states
    mu = jnp.mean(h3, axis=-1, keepdims=True)
    var = jnp.mean((h3 - mu) ** 2, axis=-1, keepdims=True)
    normed = (h3 - mu) * lax.rsqrt(var + 1e-5) * ln_weight + ln_bias
    hidden_2d = normed.reshape(T, D)
    logits = hidden_2d @ router_weight.T
    probs = jax.nn.softmax(logits, axis=-1)
    routing_weights, selected_experts = lax.top_k(probs, TOPK)
    routing_weights = routing_weights / jnp.sum(routing_weights, axis=-1,
                                                keepdims=True)

    eflat = selected_experts.reshape(-1).astype(jnp.int32)
    rwflat = routing_weights.reshape(-1)
    oh = jax.nn.one_hot(eflat, E, dtype=jnp.int32)
    csum = jnp.cumsum(oh, axis=0)
    counts = csum[-1]
    rank = jnp.take_along_axis(csum, eflat[:, None], axis=1)[:, 0] - 1
    ntiles = (counts + M - 1) // M
    tile_end = jnp.cumsum(ntiles)
    tile_start = tile_end - ntiles
    row_start = tile_start * M
    pos = jnp.take(row_start, eflat) + rank
    tile_ids = jnp.arange(NT, dtype=jnp.int32)
    te = jnp.searchsorted(tile_end, tile_ids, side='right').astype(jnp.int32)
    te = jnp.minimum(te, E - 1)
    gidx = jnp.zeros((NROWS,), jnp.int32).at[pos].set(
        jnp.arange(2 * T, dtype=jnp.int32) // TOPK)
    ws = jnp.zeros((NROWS,), jnp.float32).at[pos].set(rwflat)

    xbf = hidden_2d.astype(jnp.bfloat16)
    sc_combine = _sc_kernels()
    x_sorted = jnp.take(xbf, gidx, axis=0)

    ws3 = ws.reshape(NT, 1, M)
    act = pl.pallas_call(
        _gmm1_body,
        grid_spec=pltpu.PrefetchScalarGridSpec(
            num_scalar_prefetch=1,
            grid=(K1, NT),
            in_specs=[
                pl.BlockSpec((NROWS, D), lambda k, t, te: (0, 0)),
                pl.BlockSpec((1, F_BLK, D), lambda k, t, te: (te[t], k, 0)),
                pl.BlockSpec((1, F_BLK, D),
                             lambda k, t, te: (te[t], K1 + k, 0)),
                pl.BlockSpec((1, 1, M), lambda k, t, te: (t, 0, 0)),
            ],
            out_specs=pl.BlockSpec((M, F_BLK), lambda k, t, te: (t, k)),
        ),
        out_shape=jax.ShapeDtypeStruct((NROWS, DFF), jnp.bfloat16),
    )(te, x_sorted, gate_up_proj, gate_up_proj, ws3)

    y_sorted = pl.pallas_call(
        _gmm2_body,
        grid_spec=pltpu.PrefetchScalarGridSpec(
            num_scalar_prefetch=1,
            grid=(NT,),
            in_specs=[
                pl.BlockSpec((M, DFF), lambda t, te: (t, 0)),
                pl.BlockSpec((1, D, DFF), lambda t, te: (te[t], 0, 0)),
            ],
            out_specs=pl.BlockSpec((M, D), lambda t, te: (t, 0)),
        ),
        out_shape=jax.ShapeDtypeStruct((NROWS, D), jnp.float32),
    )(te, act, down_proj)

    pos2 = pos.reshape(T, TOPK).T.astype(jnp.int32)
    out = sc_combine(y_sorted, pos2, h3.reshape(T, D))
    return out.reshape(B, S, D)


# speedup vs baseline: 1.6624x; 1.0205x over previous
import functools

import jax
import jax.numpy as jnp
from jax import lax
from jax.experimental import pallas as pl
from jax.experimental.pallas import tpu as pltpu
from jax.experimental.pallas import tpu_sc as plsc

E = 8
TOPK = 2
D = 1024
DFF = 4096
M = 512
NT = 15
NROWS = NT * M
F_BLK = 1024
K1 = DFF // F_BLK

_NC = 2
_NS = 16
_NW = _NC * _NS
_T = 2048
_TPW = _T // _NW
_CCH = 32


@functools.cache
def _sc_kernels():
    mesh = plsc.VectorSubcoreMesh(core_axis_name="c", subcore_axis_name="s")

    @functools.partial(
        pl.kernel,
        mesh=mesh,
        out_type=jax.ShapeDtypeStruct((_T, D), jnp.float32),
        scratch_types=[
            pltpu.VMEM((2, _CCH), jnp.int32),
            pltpu.VMEM((_CCH, D), jnp.float32),
            pltpu.VMEM((_CCH, D), jnp.float32),
            pltpu.VMEM((_CCH, D), jnp.float32),
            pltpu.SemaphoreType.DMA,
        ],
    )
    def sc_combine(y_hbm, pos_hbm, h_hbm, out_hbm, idx_v, y1_v, y2_v, h_v,
                   sem):
        wid = lax.axis_index("s") * _NC + lax.axis_index("c")
        for c in range(_TPW // _CCH):
            base = wid * _TPW + c * _CCH
            pltpu.sync_copy(pos_hbm.at[0, pl.ds(base, _CCH)], idx_v.at[0])
            pltpu.sync_copy(pos_hbm.at[1, pl.ds(base, _CCH)], idx_v.at[1])
            pltpu.sync_copy(h_hbm.at[pl.ds(base, _CCH)], h_v)
            cp1 = pltpu.async_copy(y_hbm.at[idx_v.at[0]], y1_v, sem)
            cp2 = pltpu.async_copy(y_hbm.at[idx_v.at[1]], y2_v, sem)
            cp1.wait()
            cp2.wait()

            def _row(r, _):
                for j in range(D // 16):
                    sl = pl.ds(j * 16, 16)
                    h_v[r, sl] = h_v[r, sl] + y1_v[r, sl] + y2_v[r, sl]
                return 0

            lax.fori_loop(0, _CCH, _row, 0)
            pltpu.sync_copy(h_v, out_hbm.at[pl.ds(base, _CCH)])

    return sc_combine


def _gmm1_body(te_ref, x_ref, wg_ref, wu_ref, ws_ref, act_ref):
    t = pl.program_id(1)
    x = x_ref[pl.ds(t * M, M), :]
    wg = wg_ref[0].astype(jnp.bfloat16)
    wu = wu_ref[0].astype(jnp.bfloat16)
    g = lax.dot_general(x, wg, (((1,), (1,)), ((), ())),
                        preferred_element_type=jnp.float32)
    u = lax.dot_general(x, wu, (((1,), (1,)), ((), ())),
                        preferred_element_type=jnp.float32)
    act = (g * jax.nn.sigmoid(g)) * u
    w = ws_ref[0, 0, :]
    act_ref[...] = (act * w[:, None]).astype(jnp.bfloat16)


def _gmm2_body(te_ref, act_ref, wd_ref, out_ref):
    wd = wd_ref[0].astype(jnp.bfloat16)
    out_ref[...] = lax.dot_general(act_ref[...], wd,
                                   (((1,), (1,)), ((), ())),
                                   preferred_element_type=jnp.float32)


def kernel(hidden_states, residual, ln_weight, ln_bias, router_weight,
           gate_up_proj, down_proj):
    B, S, _ = hidden_states.shape
    T = B * S

    h3 = residual + hidden_states
    mu = jnp.mean(h3, axis=-1, keepdims=True)
    var = jnp.mean((h3 - mu) ** 2, axis=-1, keepdims=True)
    normed = (h3 - mu) * lax.rsqrt(var + 1e-5) * ln_weight + ln_bias
    hidden_2d = normed.reshape(T, D)
    logits = hidden_2d @ router_weight.T
    probs = jax.nn.softmax(logits, axis=-1)
    routing_weights, selected_experts = lax.top_k(probs, TOPK)
    routing_weights = routing_weights / jnp.sum(routing_weights, axis=-1,
                                                keepdims=True)

    eflat = selected_experts.reshape(-1).astype(jnp.int32)
    rwflat = routing_weights.reshape(-1)
    oh = jax.nn.one_hot(eflat, E, dtype=jnp.int32)
    csum = jnp.cumsum(oh, axis=0)
    counts = csum[-1]
    rank = jnp.take_along_axis(csum, eflat[:, None], axis=1)[:, 0] - 1
    ntiles = (counts + M - 1) // M
    tile_end = jnp.cumsum(ntiles)
    tile_start = tile_end - ntiles
    row_start = tile_start * M
    pos = jnp.take(row_start, eflat) + rank
    tile_ids = jnp.arange(NT, dtype=jnp.int32)
    te = jnp.searchsorted(tile_end, tile_ids, side='right').astype(jnp.int32)
    te = jnp.minimum(te, E - 1)
    payload = jnp.stack(
        [jnp.arange(2 * T, dtype=jnp.int32) // TOPK,
         lax.bitcast_convert_type(rwflat, jnp.int32)], axis=1)
    buf = jnp.zeros((NROWS, 2), jnp.int32).at[pos].set(payload)
    gidx = buf[:, 0]
    ws = lax.bitcast_convert_type(buf[:, 1], jnp.float32)

    xbf = hidden_2d.astype(jnp.bfloat16)
    sc_combine = _sc_kernels()
    x_sorted = jnp.take(xbf, gidx, axis=0)

    ws3 = ws.reshape(NT, 1, M)
    act = pl.pallas_call(
        _gmm1_body,
        grid_spec=pltpu.PrefetchScalarGridSpec(
            num_scalar_prefetch=1,
            grid=(K1, NT),
            in_specs=[
                pl.BlockSpec((NROWS, D), lambda k, t, te: (0, 0)),
                pl.BlockSpec((1, F_BLK, D), lambda k, t, te: (te[t], k, 0)),
                pl.BlockSpec((1, F_BLK, D),
                             lambda k, t, te: (te[t], K1 + k, 0)),
                pl.BlockSpec((1, 1, M), lambda k, t, te: (t, 0, 0)),
            ],
            out_specs=pl.BlockSpec((M, F_BLK), lambda k, t, te: (t, k)),
        ),
        out_shape=jax.ShapeDtypeStruct((NROWS, DFF), jnp.bfloat16),
    )(te, x_sorted, gate_up_proj, gate_up_proj, ws3)

    y_sorted = pl.pallas_call(
        _gmm2_body,
        grid_spec=pltpu.PrefetchScalarGridSpec(
            num_scalar_prefetch=1,
            grid=(NT,),
            in_specs=[
                pl.BlockSpec((M, DFF), lambda t, te: (t, 0)),
                pl.BlockSpec((1, D, DFF), lambda t, te: (te[t], 0, 0)),
            ],
            out_specs=pl.BlockSpec((M, D), lambda t, te: (t, 0)),
        ),
        out_shape=jax.ShapeDtypeStruct((NROWS, D), jnp.float32),
    )(te, act, down_proj)

    pos2 = pos.reshape(T, TOPK).T.astype(jnp.int32)
    out = sc_combine(y_sorted, pos2, h3.reshape(T, D))
    return out.reshape(B, S, D)
